# Initial kernel scaffold; baseline (speedup 1.0000x reference)
import jax
import jax.numpy as jnp
from jax.experimental import pallas as pl


def kernel(x, neigh_orders):
    raise NotImplementedError("write your pallas kernel here")


# speedup vs baseline: 2.9304x; 2.9304x over previous
import functools

import jax
import jax.numpy as jnp
from jax import lax
from jax.experimental import pallas as pl
from jax.experimental.pallas import tpu as pltpu
from jax.experimental.pallas import tpu_sc as plsc

NC = 2
NS = 16
NW = NC * NS
L = 16

K = 7
D = 128
CH = 16
IDXW = CH * K
CHUNKS = 82
WPN = CH * CHUNKS
PAD_NODES = NW * WPN
SCALE = 1.0 / K


def _pool_body(nn, tail, x_hbm, idx_hbm, out_hbm, idx_v, rows0, rows1,
               out_v, sem0, sem1):
    w = lax.axis_index("s") * NC + lax.axis_index("c")

    pltpu.sync_copy(idx_hbm.at[pl.ds(w * CHUNKS * IDXW, CHUNKS * IDXW)],
                    idx_v)

    def start_gather(i, rows, sem):
        pltpu.make_async_copy(x_hbm.at[idx_v.at[pl.ds(i * IDXW, IDXW)]],
                              rows, sem).start()

    def wait_gather(rows, sem):
        pltpu.make_async_copy(x_hbm.at[pl.ds(0, IDXW)], rows, sem).wait()

    def reduce_store(i, rows):
        def node(n, carry):
            r0 = n * K
            for g in range(D // L):
                sl = pl.ds(g * L, L)
                acc = rows[r0, sl]
                for j in range(1, K):
                    acc = acc + rows[r0 + j, sl]
                out_v[n, sl] = acc * SCALE
            return carry

        lax.fori_loop(0, CH, node, 0)
        base = w * WPN + i * CH
        rem = nn - base

        @pl.when(rem >= CH)
        def _():
            pltpu.sync_copy(out_v, out_hbm.at[pl.ds(base, CH)])

        @pl.when(jnp.logical_and(rem > 0, rem < CH))
        def _():
            pltpu.sync_copy(out_v.at[pl.ds(0, tail)],
                            out_hbm.at[pl.ds(base, tail)])

    start_gather(0, rows0, sem0)

    def outer(g, carry):
        i0 = g * 2
        wait_gather(rows0, sem0)
        start_gather(i0 + 1, rows1, sem1)
        reduce_store(i0, rows0)
        wait_gather(rows1, sem1)

        @pl.when(i0 + 2 < CHUNKS)
        def _():
            start_gather(i0 + 2, rows0, sem0)

        reduce_store(i0 + 1, rows1)
        return carry

    lax.fori_loop(0, CHUNKS // 2, outer, 0)


def kernel(x, neigh_orders):
    nn = (x.shape[0] + 6) // 4
    tail = nn % CH
    idx = jnp.pad(neigh_orders[: nn * K], (0, PAD_NODES * K - nn * K))

    mesh = plsc.VectorSubcoreMesh(core_axis_name="c", subcore_axis_name="s")
    pool = pl.kernel(
        functools.partial(_pool_body, nn, tail),
        mesh=mesh,
        out_type=jax.ShapeDtypeStruct((nn, D), jnp.float32),
        scratch_types=[
            pltpu.VMEM((CHUNKS * IDXW,), jnp.int32),
            pltpu.VMEM((IDXW, D), jnp.float32),
            pltpu.VMEM((IDXW, D), jnp.float32),
            pltpu.VMEM((CH, D), jnp.float32),
            pltpu.SemaphoreType.DMA,
            pltpu.SemaphoreType.DMA,
        ],
    )
    return pool(x, idx)
